# baseline (speedup 1.0000x reference)
import jax
import jax.numpy as jnp
from jax import lax
from jax.experimental import pallas as pl
from jax.experimental.pallas import tpu as pltpu
from jax.experimental.pallas import tpu_sc as plsc

_E = 16
_B = 16384
_S = 21
_NC, _NS, _L = 2, 16, 16
_NW = _NC * _NS
_BW = _B // _NW
_C = 128
_NCH = _BW // _C
_CS = _C * _S
_SB = _L


def _sc_body(tgt_hbm, oidx_hbm, in_hbm, out_hbm, scores_hbm,
             tgt_idx, oidx, t_rows, o_rows, scores, sem):
  cid = lax.axis_index("c")
  sid = lax.axis_index("s")
  wid = sid * _NC + cid
  lanes = lax.iota(jnp.int32, _L)
  e_idx = [jnp.full((_L,), e, jnp.int32) for e in range(_E)]

  for ch in range(_NCH):
    g = wid * _NCH + ch
    pltpu.sync_copy(tgt_hbm.at[g], tgt_idx)
    pltpu.sync_copy(oidx_hbm.at[g], oidx)
    cps = [pltpu.async_copy(in_hbm.at[tgt_idx.at[0]], t_rows, sem)]
    for j in range(_S):
      cps.append(pltpu.async_copy(out_hbm.at[oidx.at[j]],
                                  o_rows.at[pl.ds(j * _C, _C)], sem))
    for cp in cps:
      cp.wait()

    def block(sb, carry):
      s_loc = sb * _SB + lanes
      t_cols = [plsc.load_gather(t_rows, [s_loc, e_idx[e]]) for e in range(_E)]
      s21 = s_loc * _S
      for j in range(_S):
        kk = s21 + j
        acc = t_cols[0] * plsc.load_gather(o_rows, [kk, e_idx[0]])
        for e in range(1, _E):
          acc = acc + t_cols[e] * plsc.load_gather(o_rows, [kk, e_idx[e]])
        if j > 0:
          acc = -acc
        plsc.store_scatter(scores, [kk >> 7, kk & 127], acc)
      return carry

    lax.fori_loop(0, _C // _SB, block, 0)
    pltpu.sync_copy(scores, scores_hbm.at[g])


_NG = _NW * _NCH

_sc_scores = pl.kernel(
    _sc_body,
    out_type=jax.ShapeDtypeStruct((_NG, _S, 128), jnp.float32),
    mesh=plsc.VectorSubcoreMesh(core_axis_name="c", subcore_axis_name="s"),
    compiler_params=pltpu.CompilerParams(
        needs_layout_passes=False, use_tc_tiling_on_sc=False),
    scratch_types=[
        pltpu.VMEM((1, 128), jnp.int32),
        pltpu.VMEM((_S, 128), jnp.int32),
        pltpu.VMEM((_C, _E), jnp.float32),
        pltpu.VMEM((_CS, _E), jnp.float32),
        pltpu.VMEM((_S, 128), jnp.float32),
        pltpu.SemaphoreType.DMA,
    ],
)


_TPR = 40960


def _tp_body(src_ref, dst_ref):
  dst_ref[...] = src_ref[...].T


_tp_call = pl.pallas_call(
    _tp_body,
    grid=(pl.cdiv(1000000, _TPR),),
    in_specs=[pl.BlockSpec((_E, _TPR), lambda i: (0, i))],
    out_specs=pl.BlockSpec((_TPR, _E), lambda i: (i, 0)),
    out_shape=jax.ShapeDtypeStruct((1000000, _E), jnp.float32),
)


def _tc_body(scores_ref, out_ref):
  x = scores_ref[...]
  m = jnp.maximum(x, 0.0)
  ls = x - m - jnp.log(jnp.exp(-m) + jnp.exp(x - m))
  out_ref[...] = (-jnp.sum(ls) * (1.0 / _B))[None, None]


_tc_loss = pl.pallas_call(
    _tc_body,
    out_shape=jax.ShapeDtypeStruct((1, 1), jnp.float32),
)


def kernel(target, context, noise_words, in_table, out_table):
  tgt3d = target.astype(jnp.int32).reshape(_NG, 1, 128)
  oidx3d = jnp.concatenate(
      [context[:, None], noise_words], axis=1).astype(jnp.int32).reshape(
          _NG, _S, 128)
  in_lin = _tp_call(in_table.T)
  out_lin = _tp_call(out_table.T)
  scores = _sc_scores(tgt3d, oidx3d, in_lin, out_lin)
  loss = _tc_loss(scores.reshape(_B * _S // 128, 128))
  return loss[0, 0]


# speedup vs baseline: 1.0815x; 1.0815x over previous
import jax
import jax.numpy as jnp
from jax import lax
from jax.experimental import pallas as pl
from jax.experimental.pallas import tpu as pltpu
from jax.experimental.pallas import tpu_sc as plsc

_E = 16
_B = 16384
_S = 21
_NC, _NS, _L = 2, 16, 16
_NW = _NC * _NS
_BW = _B // _NW
_C = 128
_NCH = _BW // _C
_CS = _C * _S
_SB = _L


def _sc_body(tgt_hbm, oidx_hbm, in_hbm, out_hbm, scores_hbm,
             tgt_idx, oidx, t_rows, o_rows, scores, sem):
  cid = lax.axis_index("c")
  sid = lax.axis_index("s")
  wid = sid * _NC + cid
  lanes = lax.iota(jnp.int32, _L)
  e_idx = [jnp.full((_L,), e, jnp.int32) for e in range(_E)]

  for ch in range(_NCH):
    g = wid * _NCH + ch
    pltpu.sync_copy(tgt_hbm.at[g], tgt_idx)
    pltpu.sync_copy(oidx_hbm.at[g], oidx)
    cps = [pltpu.async_copy(in_hbm.at[tgt_idx.at[0]], t_rows, sem)]
    for j in range(_S):
      cps.append(pltpu.async_copy(out_hbm.at[oidx.at[j]],
                                  o_rows.at[pl.ds(j * _C, _C)], sem))
    for cp in cps:
      cp.wait()

    def block(sb, carry):
      s_loc = sb * _SB + lanes
      t_cols = [plsc.load_gather(t_rows, [s_loc, e_idx[e]]) for e in range(_E)]
      s21 = s_loc * _S
      for j in range(_S):
        kk = s21 + j
        acc = t_cols[0] * plsc.load_gather(o_rows, [kk, e_idx[0]])
        for e in range(1, _E):
          acc = acc + t_cols[e] * plsc.load_gather(o_rows, [kk, e_idx[e]])
        if j > 0:
          acc = -acc
        plsc.store_scatter(scores, [kk >> 7, kk & 127], acc)
      return carry

    lax.fori_loop(0, _C // _SB, block, 0)
    pltpu.sync_copy(scores, scores_hbm.at[g])


_NG = _NW * _NCH

_sc_scores = pl.kernel(
    _sc_body,
    out_type=jax.ShapeDtypeStruct((_NG, _S, 128), jnp.float32),
    mesh=plsc.VectorSubcoreMesh(core_axis_name="c", subcore_axis_name="s"),
    compiler_params=pltpu.CompilerParams(
        needs_layout_passes=False, use_tc_tiling_on_sc=False),
    scratch_types=[
        pltpu.VMEM((1, 128), jnp.int32),
        pltpu.VMEM((_S, 128), jnp.int32),
        pltpu.VMEM((_C, _E), jnp.float32),
        pltpu.VMEM((_CS, _E), jnp.float32),
        pltpu.VMEM((_S, 128), jnp.float32),
        pltpu.SemaphoreType.DMA,
    ],
)


_V = 1000000
_W = 1024
_NFULL = _V // _W
_TAIL0 = _NFULL * _W


def _rl_body(tbl_hbm, tail_hbm, lin_hbm, buf, stage, sem):
  cid = lax.axis_index("c")
  sid = lax.axis_index("s")
  wid = sid * _NC + cid
  lanes = lax.iota(jnp.int32, _L)

  def do_chunk(c0, wc, srows):
    c0 = pl.multiple_of(c0, 128)
    r0 = pl.multiple_of(c0 // 8, 8)
    cps = [
        pltpu.async_copy(tbl_hbm.at[pl.ds(0, 8), pl.ds(c0, wc)],
                         buf.at[pl.ds(0, 8), pl.ds(0, wc)], sem),
        pltpu.async_copy(tbl_hbm.at[pl.ds(8, 8), pl.ds(c0, wc)],
                         buf.at[pl.ds(8, 8), pl.ds(0, wc)], sem),
    ]
    for cp in cps:
      cp.wait()

    def row_group(r, carry):
      for a in range(8):
        col = jnp.broadcast_to(r * 8 + a, (_L,)).astype(jnp.int32)
        vals = plsc.load_gather(buf, [lanes, col])
        stage[r, pl.ds(a * _E, _E)] = vals
      return carry

    lax.fori_loop(0, srows, row_group, 0)
    pltpu.sync_copy(stage.at[pl.ds(0, srows)],
                    lin_hbm.at[pl.ds(r0, srows)])

  nch = lax.select(wid < _NFULL % _NW, _NFULL // _NW + 1, _NFULL // _NW)

  def chunk_loop(i, carry):
    do_chunk((wid + i * _NW) * _W, _W, _W // 8)
    return carry

  lax.fori_loop(0, nch, chunk_loop, 0)

  @pl.when(wid == 16)
  def _():
    do_chunk(_TAIL0, 512, 64)

  @pl.when(wid == 17)
  def _():
    pltpu.sync_copy(tail_hbm, lin_hbm.at[pl.ds(_V // 8 - 8, 8)])


_rl_call = pl.kernel(
    _rl_body,
    out_type=jax.ShapeDtypeStruct((_V // 8, 128), jnp.float32),
    mesh=plsc.VectorSubcoreMesh(core_axis_name="c", subcore_axis_name="s"),
    compiler_params=pltpu.CompilerParams(
        needs_layout_passes=False, use_tc_tiling_on_sc=True),
    scratch_types=[
        pltpu.VMEM((_E, _W), jnp.float32),
        pltpu.VMEM((_W // 8, 128), jnp.float32),
        pltpu.SemaphoreType.DMA,
    ],
)


def _tp_call(tbl):
  tail = lax.slice(tbl, (_V - 64, 0), (_V, _E)).reshape(8, 128)
  return _rl_call(tbl.T, tail).reshape(_V, _E)


def _tc_body(scores_ref, out_ref):
  x = scores_ref[...]
  m = jnp.maximum(x, 0.0)
  ls = x - m - jnp.log(jnp.exp(-m) + jnp.exp(x - m))
  out_ref[...] = (-jnp.sum(ls) * (1.0 / _B))[None, None]


_tc_loss = pl.pallas_call(
    _tc_body,
    out_shape=jax.ShapeDtypeStruct((1, 1), jnp.float32),
)


def kernel(target, context, noise_words, in_table, out_table):
  tgt3d = target.astype(jnp.int32).reshape(_NG, 1, 128)
  oidx3d = jnp.concatenate(
      [context[:, None], noise_words], axis=1).astype(jnp.int32).reshape(
          _NG, _S, 128)
  in_lin = _tp_call(in_table)
  out_lin = _tp_call(out_table)
  scores = _sc_scores(tgt3d, oidx3d, in_lin, out_lin)
  loss = _tc_loss(scores.reshape(_B * _S // 128, 128))
  return loss[0, 0]


# speedup vs baseline: 3.1808x; 2.9411x over previous
import jax
import jax.numpy as jnp
from jax import lax
from jax.experimental import pallas as pl
from jax.experimental.pallas import tpu as pltpu
from jax.experimental.pallas import tpu_sc as plsc

_E = 16
_B = 16384
_S = 21
_NC, _NS, _L = 2, 16, 16
_NW = _NC * _NS
_BW = _B // _NW
_C = 128
_NCH = _BW // _C
_CS = _C * _S
_SB = _L


def _sc_body(tgt_hbm, oidx_hbm, in_hbm, out_hbm, scores_hbm,
             tgt_idx, oidx, t_rows, o_rows, scores, sem):
  cid = lax.axis_index("c")
  sid = lax.axis_index("s")
  wid = sid * _NC + cid
  lanes = lax.iota(jnp.int32, _L)
  e_idx = [jnp.full((_L,), e, jnp.int32) for e in range(_E)]

  for ch in range(_NCH):
    g = wid * _NCH + ch
    pltpu.sync_copy(tgt_hbm.at[g], tgt_idx)
    pltpu.sync_copy(oidx_hbm.at[g], oidx)
    cps = [pltpu.async_copy(in_hbm.at[tgt_idx.at[0]], t_rows, sem)]
    for j in range(_S):
      cps.append(pltpu.async_copy(out_hbm.at[oidx.at[j]],
                                  o_rows.at[pl.ds(j * _C, _C)], sem))
    for cp in cps:
      cp.wait()

    def block(sb, carry):
      s_loc = sb * _SB + lanes
      t_cols = [plsc.load_gather(t_rows, [s_loc, e_idx[e]]) for e in range(_E)]
      s21 = s_loc * _S
      for j in range(_S):
        kk = s21 + j
        acc = t_cols[0] * plsc.load_gather(o_rows, [kk, e_idx[0]])
        for e in range(1, _E):
          acc = acc + t_cols[e] * plsc.load_gather(o_rows, [kk, e_idx[e]])
        if j > 0:
          acc = -acc
        plsc.store_scatter(scores, [kk >> 7, kk & 127], acc)
      return carry

    lax.fori_loop(0, _C // _SB, block, 0)
    pltpu.sync_copy(scores, scores_hbm.at[g])


_NG = _NW * _NCH

_sc_scores = pl.kernel(
    _sc_body,
    out_type=jax.ShapeDtypeStruct((_NG, _S, 128), jnp.float32),
    mesh=plsc.VectorSubcoreMesh(core_axis_name="c", subcore_axis_name="s"),
    compiler_params=pltpu.CompilerParams(
        needs_layout_passes=False, use_tc_tiling_on_sc=False),
    scratch_types=[
        pltpu.VMEM((1, 128), jnp.int32),
        pltpu.VMEM((_S, 128), jnp.int32),
        pltpu.VMEM((_C, _E), jnp.float32),
        pltpu.VMEM((_CS, _E), jnp.float32),
        pltpu.VMEM((_S, 128), jnp.float32),
        pltpu.SemaphoreType.DMA,
    ],
)


_V = 1000000
_W = 1024
_NFULL = _V // _W
_TAIL0 = _NFULL * _W


def _rl_body(tbl_hbm, tail_hbm, lin_hbm, buf0, buf1, st0, st1, sem0, sem1,
             osem0, osem1):
  cid = lax.axis_index("c")
  sid = lax.axis_index("s")
  wid = sid * _NC + cid
  lanes = lax.iota(jnp.int32, _L)
  bufs = (buf0, buf1)
  stages = (st0, st1)
  isems = (sem0, sem1)
  osems = (osem0, osem1)

  def chunk_c0(slot):
    ch = wid + slot * _NW
    ch = lax.select(ch < _NFULL, ch, wid)
    return pl.multiple_of(ch * _W, 128)

  def in_copies(slot, p):
    c0 = chunk_c0(slot)
    return [
        pltpu.make_async_copy(tbl_hbm.at[pl.ds(0, 8), pl.ds(c0, _W)],
                              bufs[p].at[pl.ds(0, 8)], isems[p]),
        pltpu.make_async_copy(tbl_hbm.at[pl.ds(8, 8), pl.ds(c0, _W)],
                              bufs[p].at[pl.ds(8, 8)], isems[p]),
    ]

  def out_copy(slot, p):
    o0 = pl.multiple_of(chunk_c0(slot) * _E, 128)
    return pltpu.make_async_copy(stages[p], lin_hbm.at[pl.ds(o0, _W * _E)],
                                 osems[p])

  lanes16 = lanes * _E

  def shuffle(p):
    def tile_body(t, carry):
      for jb in range(8):
        for e in range(_E):
          vals = bufs[p][e, pl.ds(t * 128 + jb * _E, _E)]
          idx = lanes16 + (t * (128 * _E) + jb * (_E * _E) + e)
          plsc.store_scatter(stages[p], [idx], vals)
      return carry
    lax.fori_loop(0, 8, tile_body, 0)

  for p in (0, 1):
    for cp in in_copies(p, p):
      cp.start()

  def slot_pair(ii, carry):
    for p in (0, 1):
      slot = ii * 2 + p
      for cp in in_copies(slot, p):
        cp.wait()

      @pl.when(ii > 0)
      def _():
        out_copy(slot - 2, p).wait()

      shuffle(p)
      out_copy(slot, p).start()

      @pl.when(slot + 2 < 32)
      def _():
        for cp in in_copies(slot + 2, p):
          cp.start()
    return carry

  lax.fori_loop(0, 16, slot_pair, 0)
  for p in (0, 1):
    out_copy(30 + p, p).wait()

  @pl.when(wid == 16)
  def _():
    c0 = _TAIL0
    cps = [
        pltpu.make_async_copy(tbl_hbm.at[pl.ds(0, 8), pl.ds(c0, 512)],
                              buf0.at[pl.ds(0, 8), pl.ds(0, 512)], sem0),
        pltpu.make_async_copy(tbl_hbm.at[pl.ds(8, 8), pl.ds(c0, 512)],
                              buf0.at[pl.ds(8, 8), pl.ds(0, 512)], sem0),
    ]
    for cp in cps:
      cp.start()
    for cp in cps:
      cp.wait()

    def tile_body(t, carry):
      for jb in range(8):
        for e in range(_E):
          vals = buf0[e, pl.ds(t * 128 + jb * _E, _E)]
          idx = lanes16 + (t * (128 * _E) + jb * (_E * _E) + e)
          plsc.store_scatter(st0, [idx], vals)
      return carry

    lax.fori_loop(0, 4, tile_body, 0)
    pltpu.sync_copy(st0.at[pl.ds(0, 512 * _E)],
                    lin_hbm.at[pl.ds(_TAIL0 * _E, 512 * _E)])

  @pl.when(wid == 17)
  def _():
    pltpu.sync_copy(tail_hbm, lin_hbm.at[pl.ds((_V - 64) * _E, 64 * _E)])


_rl_call = pl.kernel(
    _rl_body,
    out_type=jax.ShapeDtypeStruct((_V * _E,), jnp.float32),
    mesh=plsc.VectorSubcoreMesh(core_axis_name="c", subcore_axis_name="s"),
    compiler_params=pltpu.CompilerParams(
        needs_layout_passes=False, use_tc_tiling_on_sc=True),
    scratch_types=[
        pltpu.VMEM((_E, _W), jnp.float32),
        pltpu.VMEM((_E, _W), jnp.float32),
        pltpu.VMEM((_W * _E,), jnp.float32),
        pltpu.VMEM((_W * _E,), jnp.float32),
        pltpu.SemaphoreType.DMA,
        pltpu.SemaphoreType.DMA,
        pltpu.SemaphoreType.DMA,
        pltpu.SemaphoreType.DMA,
    ],
)


def _tp_call(tbl):
  tail = lax.slice(tbl, (_V - 64, 0), (_V, _E)).reshape(64 * _E)
  return _rl_call(tbl.T, tail).reshape(_V, _E)


def _tc_body(scores_ref, out_ref):
  x = scores_ref[...]
  m = jnp.maximum(x, 0.0)
  ls = x - m - jnp.log(jnp.exp(-m) + jnp.exp(x - m))
  out_ref[...] = (-jnp.sum(ls) * (1.0 / _B))[None, None]


_tc_loss = pl.pallas_call(
    _tc_body,
    out_shape=jax.ShapeDtypeStruct((1, 1), jnp.float32),
)


def kernel(target, context, noise_words, in_table, out_table):
  tgt3d = target.astype(jnp.int32).reshape(_NG, 1, 128)
  oidx3d = jnp.concatenate(
      [context[:, None], noise_words], axis=1).astype(jnp.int32).reshape(
          _NG, _S, 128)
  in_lin = _tp_call(in_table)
  out_lin = _tp_call(out_table)
  scores = _sc_scores(tgt3d, oidx3d, in_lin, out_lin)
  loss = _tc_loss(scores.reshape(_B * _S // 128, 128))
  return loss[0, 0]


# speedup vs baseline: 3.3424x; 1.0508x over previous
import jax
import jax.numpy as jnp
from jax import lax
from jax.experimental import pallas as pl
from jax.experimental.pallas import tpu as pltpu
from jax.experimental.pallas import tpu_sc as plsc

_E = 16
_B = 16384
_S = 21
_NC, _NS, _L = 2, 16, 16
_NW = _NC * _NS
_BW = _B // _NW
_C = 128
_NCH = _BW // _C
_CS = _C * _S
_SB = _L


def _sc_body(tgt_hbm, oidx_hbm, in_hbm, out_hbm, scores_hbm,
             tgt_idx0, tgt_idx1, oidx0, oidx1, t_rows0, t_rows1,
             o_rows0, o_rows1, scores0, scores1, sem0, sem1):
  cid = lax.axis_index("c")
  sid = lax.axis_index("s")
  wid = sid * _NC + cid
  lanes = lax.iota(jnp.int32, _L)
  e_idx = [jnp.full((_L,), e, jnp.int32) for e in range(_E)]
  tgt_idx = (tgt_idx0, tgt_idx1)
  oidx = (oidx0, oidx1)
  t_rows = (t_rows0, t_rows1)
  o_rows = (o_rows0, o_rows1)
  scores = (scores0, scores1)
  sems = (sem0, sem1)

  def stage_idx(ch, p):
    g = wid * _NCH + ch
    pltpu.sync_copy(tgt_hbm.at[g], tgt_idx[p])
    pltpu.sync_copy(oidx_hbm.at[g], oidx[p])

  def start_gathers(p):
    cps = [pltpu.async_copy(in_hbm.at[tgt_idx[p].at[0]], t_rows[p], sems[p])]
    for j in range(_S):
      cps.append(pltpu.async_copy(out_hbm.at[oidx[p].at[j]],
                                  o_rows[p].at[pl.ds(j * _C, _C)], sems[p]))
    return cps

  cps = {}
  for p in (0, 1):
    stage_idx(p, p)
    cps[p] = start_gathers(p)

  for ch in range(_NCH):
    p = ch % 2
    for cp in cps[p]:
      cp.wait()

    def block(sb, carry):
      s_loc = sb * _SB + lanes
      t_cols = [plsc.load_gather(t_rows[p], [s_loc, e_idx[e]])
                for e in range(_E)]
      s21 = s_loc * _S
      for j in range(_S):
        kk = s21 + j
        acc = t_cols[0] * plsc.load_gather(o_rows[p], [kk, e_idx[0]])
        for e in range(1, _E):
          acc = acc + t_cols[e] * plsc.load_gather(o_rows[p], [kk, e_idx[e]])
        if j > 0:
          acc = -acc
        plsc.store_scatter(scores[p], [kk >> 7, kk & 127], acc)
      return carry

    lax.fori_loop(0, _C // _SB, block, 0)
    pltpu.sync_copy(scores[p], scores_hbm.at[wid * _NCH + ch])
    if ch + 2 < _NCH:
      stage_idx(ch + 2, p)
      cps[p] = start_gathers(p)


_NG = _NW * _NCH

_sc_scores = pl.kernel(
    _sc_body,
    out_type=jax.ShapeDtypeStruct((_NG, _S, 128), jnp.float32),
    mesh=plsc.VectorSubcoreMesh(core_axis_name="c", subcore_axis_name="s"),
    compiler_params=pltpu.CompilerParams(
        needs_layout_passes=False, use_tc_tiling_on_sc=False),
    scratch_types=[
        pltpu.VMEM((1, 128), jnp.int32),
        pltpu.VMEM((1, 128), jnp.int32),
        pltpu.VMEM((_S, 128), jnp.int32),
        pltpu.VMEM((_S, 128), jnp.int32),
        pltpu.VMEM((_C, _E), jnp.float32),
        pltpu.VMEM((_C, _E), jnp.float32),
        pltpu.VMEM((_CS, _E), jnp.float32),
        pltpu.VMEM((_CS, _E), jnp.float32),
        pltpu.VMEM((_S, 128), jnp.float32),
        pltpu.VMEM((_S, 128), jnp.float32),
        pltpu.SemaphoreType.DMA,
        pltpu.SemaphoreType.DMA,
    ],
)


_V = 1000000
_W = 1024
_NFULL = _V // _W
_TAIL0 = _NFULL * _W


def _rl_body(tbl_hbm_a, tbl_hbm_b, tail_hbm_a, tail_hbm_b,
             lin_hbm_a, lin_hbm_b, buf0, buf1, st0, st1, sem0, sem1,
             osem0, osem1):
  cid = lax.axis_index("c")
  sid = lax.axis_index("s")
  wid = sid * _NC + cid
  lanes = lax.iota(jnp.int32, _L)
  bufs = (buf0, buf1)
  stages = (st0, st1)
  isems = (sem0, sem1)
  osems = (osem0, osem1)
  lanes16 = lanes * _E

  def chunk_c0(slot):
    ch = wid + slot * _NW
    ch = lax.select(ch < _NFULL, ch, wid)
    return pl.multiple_of(ch * _W, 128)

  def in_copies(tbl_hbm, slot, p):
    c0 = chunk_c0(slot)
    return [
        pltpu.make_async_copy(tbl_hbm.at[:, pl.ds(c0, _W)], bufs[p],
                              isems[p]),
    ]

  def out_copy(lin_hbm, slot, p):
    o0 = pl.multiple_of(chunk_c0(slot) * _E, 128)
    return pltpu.make_async_copy(stages[p], lin_hbm.at[pl.ds(o0, _W * _E)],
                                 osems[p])

  def shuffle(p):
    def tile_body(t, carry):
      for jb in range(8):
        for e in range(_E):
          vals = bufs[p][e, pl.ds(t * 128 + jb * _E, _E)]
          idx = lanes16 + (t * (128 * _E) + jb * (_E * _E) + e)
          plsc.store_scatter(stages[p], [idx], vals)
      return carry
    lax.fori_loop(0, 8, tile_body, 0)

  def prologue(tbl_hbm):
    for p in (0, 1):
      for cp in in_copies(tbl_hbm, p, p):
        cp.start()

  def main_loop(tbl_hbm, lin_hbm, first):
    def slot_pair(ii, carry):
      for p in (0, 1):
        slot = ii * 2 + p
        for cp in in_copies(tbl_hbm, slot, p):
          cp.wait()

        if first:
          @pl.when(ii > 0)
          def _():
            out_copy(lin_hbm, slot - 2, p).wait()
        else:
          out_copy(lin_hbm, slot - 2, p).wait()

        shuffle(p)
        out_copy(lin_hbm, slot, p).start()

        @pl.when(slot + 2 < 32)
        def _():
          for cp in in_copies(tbl_hbm, slot + 2, p):
            cp.start()
      return carry
    lax.fori_loop(0, 16, slot_pair, 0)

  def tails(tbl_hbm, tail_hbm, lin_hbm):
    @pl.when(wid == 16)
    def _():
      c0 = _TAIL0
      cps = [
          pltpu.make_async_copy(tbl_hbm.at[:, pl.ds(c0, 512)],
                                buf0.at[:, pl.ds(0, 512)], sem0),
      ]
      for cp in cps:
        cp.start()
      for cp in cps:
        cp.wait()

      def tile_body(t, carry):
        for jb in range(8):
          for e in range(_E):
            vals = buf0[e, pl.ds(t * 128 + jb * _E, _E)]
            idx = lanes16 + (t * (128 * _E) + jb * (_E * _E) + e)
            plsc.store_scatter(st0, [idx], vals)
        return carry

      lax.fori_loop(0, 4, tile_body, 0)
      pltpu.sync_copy(st0.at[pl.ds(0, 512 * _E)],
                      lin_hbm.at[pl.ds(_TAIL0 * _E, 512 * _E)])

    @pl.when(wid == 17)
    def _():
      pltpu.sync_copy(tail_hbm, lin_hbm.at[pl.ds((_V - 64) * _E, 64 * _E)])

  prologue(tbl_hbm_a)
  main_loop(tbl_hbm_a, lin_hbm_a, True)
  for p in (0, 1):
    out_copy(lin_hbm_a, 30 + p, p).wait()
  prologue(tbl_hbm_b)
  main_loop(tbl_hbm_b, lin_hbm_b, True)
  for p in (0, 1):
    out_copy(lin_hbm_b, 30 + p, p).wait()
  tails(tbl_hbm_a, tail_hbm_a, lin_hbm_a)
  tails(tbl_hbm_b, tail_hbm_b, lin_hbm_b)


_rl_call = pl.kernel(
    _rl_body,
    out_type=(jax.ShapeDtypeStruct((_V * _E,), jnp.float32),
              jax.ShapeDtypeStruct((_V * _E,), jnp.float32)),
    mesh=plsc.VectorSubcoreMesh(core_axis_name="c", subcore_axis_name="s"),
    compiler_params=pltpu.CompilerParams(
        needs_layout_passes=False, use_tc_tiling_on_sc=True),
    scratch_types=[
        pltpu.VMEM((_E, _W), jnp.float32),
        pltpu.VMEM((_E, _W), jnp.float32),
        pltpu.VMEM((_W * _E,), jnp.float32),
        pltpu.VMEM((_W * _E,), jnp.float32),
        pltpu.SemaphoreType.DMA,
        pltpu.SemaphoreType.DMA,
        pltpu.SemaphoreType.DMA,
        pltpu.SemaphoreType.DMA,
    ],
)


def _relayout_tables(in_table, out_table):
  tail_a = lax.slice(in_table, (_V - 64, 0), (_V, _E)).reshape(64 * _E)
  tail_b = lax.slice(out_table, (_V - 64, 0), (_V, _E)).reshape(64 * _E)
  lin_a, lin_b = _rl_call(in_table.T, out_table.T, tail_a, tail_b)
  return lin_a.reshape(_V, _E), lin_b.reshape(_V, _E)


def _tc_body(scores_ref, out_ref):
  x = scores_ref[...]
  m = jnp.maximum(x, 0.0)
  ls = x - m - jnp.log(jnp.exp(-m) + jnp.exp(x - m))
  out_ref[...] = (-jnp.sum(ls) * (1.0 / _B))[None, None]


_tc_loss = pl.pallas_call(
    _tc_body,
    out_shape=jax.ShapeDtypeStruct((1, 1), jnp.float32),
)


def kernel(target, context, noise_words, in_table, out_table):
  tgt3d = target.astype(jnp.int32).reshape(_NG, 1, 128)
  oidx3d = jnp.concatenate(
      [context[:, None], noise_words], axis=1).astype(jnp.int32).reshape(
          _NG, _S, 128)
  in_lin, out_lin = _relayout_tables(in_table, out_table)
  scores = _sc_scores(tgt3d, oidx3d, in_lin, out_lin)
  loss = _tc_loss(scores.reshape(_B * _S // 128, 128))
  return loss[0, 0]
